# baseline (speedup 1.0000x reference)
import functools

import jax
import jax.numpy as jnp
from jax import lax
from jax.experimental import pallas as pl
from jax.experimental.pallas import tpu as pltpu
from jax.experimental.pallas import tpu_sc as plsc


def _make_sc_gather(vocab, d, n_tokens, dtype):
    info = plsc.get_sparse_core_info()
    nc, ns = info.num_cores, info.num_subcores
    nw = nc * ns
    assert n_tokens % nw == 0
    b_per_w = n_tokens // nw

    mesh = plsc.VectorSubcoreMesh(core_axis_name="c", subcore_axis_name="s")

    @functools.partial(
        pl.kernel,
        mesh=mesh,
        compiler_params=pltpu.CompilerParams(use_tc_tiling_on_sc=False),
        out_type=jax.ShapeDtypeStruct((n_tokens, d), dtype),
        scratch_types=[
            pltpu.VMEM((b_per_w,), jnp.int32),
            pltpu.VMEM((b_per_w, d), dtype),
            pltpu.SemaphoreType.DMA,
        ],
    )
    def gather_kernel(table_hbm, idx_hbm, out_hbm, idx_v, rows_v, sem):
        wid = lax.axis_index("s") * nc + lax.axis_index("c")
        base = wid * b_per_w
        pltpu.sync_copy(idx_hbm.at[pl.ds(base, b_per_w)], idx_v)
        pltpu.async_copy(table_hbm.at[idx_v], rows_v, sem).wait()
        pltpu.sync_copy(rows_v, out_hbm.at[pl.ds(base, b_per_w)])

    return gather_kernel


def _fused_table(tgt_embs, logits):
    v, d = tgt_embs.shape
    n = logits.shape[1]

    def body(t_ref, l_ref, o_ref):
        o_ref[...] = jnp.dot(
            t_ref[...], l_ref[...], preferred_element_type=jnp.float32
        ).astype(jnp.bfloat16)

    return pl.pallas_call(
        body,
        out_shape=jax.ShapeDtypeStruct((v, n), jnp.bfloat16),
    )(tgt_embs, logits)


def _logits_lookup(idx_flat, fused, block_b=512):
    t = idx_flat.shape[0]
    v, n = fused.shape
    assert t % block_b == 0

    def body(idx_ref, fused_ref, out_ref):
        idx = idx_ref[...]
        oh = (idx == lax.broadcasted_iota(jnp.int32, (block_b, v), 1)).astype(
            jnp.bfloat16
        )
        out_ref[...] = jnp.dot(oh, fused_ref[...], preferred_element_type=jnp.float32)

    return pl.pallas_call(
        body,
        grid=(t // block_b,),
        in_specs=[
            pl.BlockSpec((block_b, 1), lambda i: (i, 0)),
            pl.BlockSpec((v, n), lambda i: (0, 0)),
        ],
        out_specs=pl.BlockSpec((block_b, n), lambda i: (i, 0)),
        out_shape=jax.ShapeDtypeStruct((t, n), jnp.float32),
    )(idx_flat.reshape(t, 1), fused)


def kernel(source_enumerate, target_enumerate, src_embs, tgt_embs, logits):
    b, s = source_enumerate.shape
    t = b * s
    src_v, d = src_embs.shape
    tgt_v = tgt_embs.shape[0]
    n = logits.shape[1]

    src_idx = source_enumerate.reshape(t).astype(jnp.int32)
    tgt_idx = target_enumerate.reshape(t).astype(jnp.int32)

    src_emb = _make_sc_gather(src_v, d, t, src_embs.dtype)(src_embs, src_idx)
    tgt_emb = _make_sc_gather(tgt_v, d, t, tgt_embs.dtype)(tgt_embs, tgt_idx)

    fused = _fused_table(tgt_embs, logits)
    out_logits = _logits_lookup(tgt_idx, fused)

    return (
        src_emb.reshape(b, s, d),
        tgt_emb.reshape(b, s, d),
        out_logits.reshape(b, s, n),
    )
